# baseline (speedup 1.0000x reference)
import functools

import jax
import jax.numpy as jnp
from jax import lax
from jax.experimental import pallas as pl
from jax.experimental.pallas import tpu as pltpu
from jax.experimental.pallas import tpu_sc as plsc

NEG = -1.0e30
_B, _MAXT, _MAXU, _A = 8, 200, 51, 512
_ED, _UD = 256, 64
_NC, _NS, _L = 2, 16, 16
_NW = _NC * _NS
_ROWS_PER_W = _ED // _NW
_ROWLEN = _B * _UD
_CHUNK = _ROWS_PER_W * _ROWLEN
_GCH = 128


def _sc_body(lp_hbm, labels_hbm, outB_hbm, outL_hbm, idx_v, rows_v, lab_v, sem):
    wid = lax.axis_index("s") * _NC + lax.axis_index("c")

    for b in range(_B):
        pltpu.sync_copy(labels_hbm.at[b], lab_v.at[b, pl.ds(0, _MAXU - 1)])

    def do_array(out_hbm, is_label):
        maxu = (_MAXU - 1) if is_label else _MAXU
        lane = lax.broadcasted_iota(jnp.int32, (_L,), 0)

        def coords(i):
            d = wid * _ROWS_PER_W + i // (_ROWLEN // _L)
            r = i % (_ROWLEN // _L)
            b = r // (_UD // _L)
            u = (r % (_UD // _L)) * _L + lane
            t = d - u
            valid = (t >= 0) & (t < _MAXT) & (u < maxu)
            return b, u, t, valid

        def build(i, _):
            b, u, t, valid = coords(i)
            tc = jnp.clip(t, 0, _MAXT - 1)
            uc = jnp.clip(u, 0, _MAXU - 1)
            if is_label:
                sel = jnp.where(u < _MAXU - 1,
                                lab_v[b, pl.ds((i % 4) * _L, _L)], 0)
            else:
                sel = 0
            idx = ((b * _MAXT + tc) * _MAXU + uc) * _A + sel
            idx_v[pl.ds(i * _L, _L)] = idx
            return 0

        lax.fori_loop(0, _CHUNK // _L, build, 0, unroll=4)

        copies = [
            pltpu.make_async_copy(
                lp_hbm.at[idx_v.at[pl.ds(k * _GCH, _GCH)]],
                rows_v.at[pl.ds(k * _GCH, _GCH)],
                sem,
            )
            for k in range(_CHUNK // _GCH)
        ]
        for c in copies:
            c.start()
        for c in copies:
            c.wait()

        def fix(i, _):
            _, _, _, valid = coords(i)
            x = rows_v[pl.ds(i * _L, _L)]
            rows_v[pl.ds(i * _L, _L)] = jnp.where(valid, x, NEG)
            return 0

        lax.fori_loop(0, _CHUNK // _L, fix, 0, unroll=4)
        pltpu.sync_copy(rows_v, out_hbm.at[pl.ds(wid * _CHUNK, _CHUNK)])

    do_array(outB_hbm, False)
    do_array(outL_hbm, True)


@functools.partial(
    pl.kernel,
    mesh=plsc.VectorSubcoreMesh(core_axis_name="c", subcore_axis_name="s"),
    out_type=[
        jax.ShapeDtypeStruct((_ED * _B * _UD,), jnp.float32),
        jax.ShapeDtypeStruct((_ED * _B * _UD,), jnp.float32),
    ],
    scratch_types=[
        pltpu.VMEM((_CHUNK,), jnp.int32),
        pltpu.VMEM((_CHUNK,), jnp.float32),
        pltpu.VMEM((_B, _UD), jnp.int32),
        pltpu.SemaphoreType.DMA,
    ],
)
def _sc_gather(lp_hbm, labels_hbm, outB_hbm, outL_hbm, idx_v, rows_v, lab_v, sem):
    _sc_body(lp_hbm, labels_hbm, outB_hbm, outL_hbm, idx_v, rows_v, lab_v, sem)


def _lae(x, y):
    m = jnp.maximum(x, y)
    return m + jnp.log1p(jnp.exp(-jnp.abs(x - y)))


def _dp_kernel(skewB_ref, skewL_ref, dstar_ref, umat_ref, out_ref):
    iota_u = jax.lax.broadcasted_iota(jnp.int32, (_B, _UD), 1)
    dstar = dstar_ref[...]
    mask_u = iota_u == umat_ref[...]

    a0 = jnp.where((iota_u == 0), 0.0, NEG).astype(jnp.float32)
    acc0 = jnp.zeros((_B, _UD), jnp.float32)
    negcol = jnp.full((_B, 1), NEG, jnp.float32)

    def body(d, carry):
        a, acc = carry
        bv = skewB_ref[d - 1]
        lv = skewL_ref[d - 1]
        c = a + lv
        shifted = jnp.concatenate([negcol, c[:, : _UD - 1]], axis=1)
        a_new = _lae(a + bv, shifted)
        bd = skewB_ref[d]
        hit = mask_u & (dstar == d)
        acc = acc + jnp.where(hit, a_new + bd, 0.0)
        return a_new, acc

    _, acc = jax.lax.fori_loop(1, _MAXT + _MAXU - 1, body, (a0, acc0))
    out_ref[...] = -jnp.sum(acc, keepdims=True) / _B


@jax.jit
def kernel(log_probs, labels, T, U):
    lp_flat = log_probs.reshape(-1)
    skewB_flat, skewL_flat = _sc_gather(lp_flat, labels.astype(jnp.int32))
    skewB = skewB_flat.reshape(_ED, _B, _UD)
    skewL = skewL_flat.reshape(_ED, _B, _UD)

    dstar = (T + U - 1).astype(jnp.int32)
    dstar_mat = jnp.broadcast_to(dstar[:, None], (_B, _UD))
    umat = jnp.broadcast_to(U.astype(jnp.int32)[:, None], (_B, _UD))

    out = pl.pallas_call(
        _dp_kernel,
        out_shape=jax.ShapeDtypeStruct((1, 1), jnp.float32),
        in_specs=[
            pl.BlockSpec(memory_space=pltpu.VMEM),
            pl.BlockSpec(memory_space=pltpu.VMEM),
            pl.BlockSpec(memory_space=pltpu.VMEM),
            pl.BlockSpec(memory_space=pltpu.VMEM),
        ],
        out_specs=pl.BlockSpec(memory_space=pltpu.VMEM),
    )(skewB, skewL, dstar_mat, umat)
    return out[0, 0]
